# baseline (speedup 1.0000x reference)
import functools

import jax
import jax.numpy as jnp
from jax import lax
from jax.experimental import pallas as pl
from jax.experimental.pallas import tpu as pltpu
from jax.experimental.pallas import tpu_sc as plsc

_B, _N, _D, _R, _C = 64, 8192, 128, 5, 5
_L = 16
_NW = 32
_NSPLIT = 1
_BC = _B // _NSPLIT
_RPW = _BC // _NW


_RB = 2


def _score_body(x_ref, w_ref, o_ref):
    w = w_ref[...]
    for r in range(_RB):
        o_ref[r, 0, :] = lax.dot_general(
            w, x_ref[r], (((1,), (1,)), ((), ())),
            preferred_element_type=jnp.float32)[0]


def _scores(x, w1row, off):
    return pl.pallas_call(
        _score_body,
        grid=(_BC // _RB,),
        in_specs=[
            pl.BlockSpec((_RB, _N, _D), lambda b: (b + off, 0, 0)),
            pl.BlockSpec((1, _D), lambda b: (0, 0)),
        ],
        out_specs=pl.BlockSpec((_RB, 1, _N), lambda b: (b, 0, 0)),
        out_shape=jax.ShapeDtypeStruct((_BC, 1, _N), jnp.float32),
    )(x, w1row)


def _insert(lists, v):
    t0, t1, t2, t3, t4, u0, u1, u2, u3, u4 = lists
    a = v
    m = jnp.maximum(t0, a); a = jnp.minimum(t0, a); t0 = m
    m = jnp.maximum(t1, a); a = jnp.minimum(t1, a); t1 = m
    m = jnp.maximum(t2, a); a = jnp.minimum(t2, a); t2 = m
    m = jnp.maximum(t3, a); a = jnp.minimum(t3, a); t3 = m
    t4 = jnp.maximum(t4, a)
    b = v
    m = jnp.minimum(u0, b); b = jnp.maximum(u0, b); u0 = m
    m = jnp.minimum(u1, b); b = jnp.maximum(u1, b); u1 = m
    m = jnp.minimum(u2, b); b = jnp.maximum(u2, b); u2 = m
    m = jnp.minimum(u3, b); b = jnp.maximum(u3, b); u3 = m
    u4 = jnp.minimum(u4, b)
    return (t0, t1, t2, t3, t4, u0, u1, u2, u3, u4)


def _extract_feat(lists, lane, neg, pos, zeros):
    t0, t1, t2, t3, t4, u0, u1, u2, u3, u4 = lists
    fv = jnp.where(lane == 2 * _R, jnp.float32(1.0), zeros)
    for k in range(_R):
        m = jnp.max(t0)
        fv = jnp.where(lane == k, m, fv)
        sel = lane == plsc.all_reduce_ffs(t0 == m)
        t0 = jnp.where(sel, t1, t0)
        t1 = jnp.where(sel, t2, t1)
        t2 = jnp.where(sel, t3, t2)
        t3 = jnp.where(sel, t4, t3)
        t4 = jnp.where(sel, neg, t4)
    for k in range(_R):
        m = jnp.min(u0)
        fv = jnp.where(lane == 2 * _R - 1 - k, m, fv)
        sel = lane == plsc.all_reduce_ffs(u0 == m)
        u0 = jnp.where(sel, u1, u0)
        u1 = jnp.where(sel, u2, u1)
        u2 = jnp.where(sel, u3, u2)
        u3 = jnp.where(sel, u4, u3)
        u4 = jnp.where(sel, pos, u4)
    return fv


_NHALF = _N // 2


def _sc_topk_body(s_hbm, w2_hbm, out_hbm, s_v, w2_v, out_v,
                  sem0, sem1, sem2, sem3):
    cid = lax.axis_index("c")
    sid = lax.axis_index("s")
    wid = sid * 2 + cid
    base = wid * (_RPW * _N)
    c0 = pltpu.async_copy(s_hbm.at[pl.ds(base, _NHALF)],
                          s_v.at[pl.ds(0, _NHALF)], sem0)
    c1 = pltpu.async_copy(s_hbm.at[pl.ds(base + _N, _NHALF)],
                          s_v.at[pl.ds(_N, _NHALF)], sem1)
    c2 = pltpu.async_copy(s_hbm.at[pl.ds(base + _NHALF, _NHALF)],
                          s_v.at[pl.ds(_NHALF, _NHALF)], sem2)
    c3 = pltpu.async_copy(s_hbm.at[pl.ds(base + _N + _NHALF, _NHALF)],
                          s_v.at[pl.ds(_N + _NHALF, _NHALF)], sem3)

    neg = jnp.full((_L,), -jnp.inf, jnp.float32)
    pos = jnp.full((_L,), jnp.inf, jnp.float32)
    zeros = jnp.zeros((_L,), jnp.float32)
    lane = lax.iota(jnp.int32, _L)

    def step(i, carry):
        r0, r1 = carry
        v0 = s_v[pl.ds(i * _L, _L)]
        v1 = s_v[pl.ds(_N + i * _L, _L)]
        return (_insert(r0, v0), _insert(r1, v1))

    init = ((neg,) * 5 + (pos,) * 5, (neg,) * 5 + (pos,) * 5)
    c0.wait()
    c1.wait()
    carry = lax.fori_loop(0, _NHALF // _L, step, init, unroll=8)
    c2.wait()
    c3.wait()
    carry = lax.fori_loop(_NHALF // _L, _N // _L, step, carry, unroll=8)
    pltpu.sync_copy(w2_hbm, w2_v)

    for rl in range(_RPW):
        fv = _extract_feat(carry[rl], lane, neg, pos, zeros)
        ov = zeros
        for c in range(_C):
            w = w2_v[pl.ds(c * _L, _L)]
            ov = jnp.where(lane == c, jnp.sum(fv * w), ov)
        out_v[pl.ds(rl * _L, _L)] = ov

    pltpu.sync_copy(out_v, out_hbm.at[pl.ds(wid * (_RPW * _L), _RPW * _L)])


@functools.cache
def _sc_topk():
    return functools.partial(
        pl.kernel,
        mesh=plsc.VectorSubcoreMesh(core_axis_name="c", subcore_axis_name="s"),
        compiler_params=pltpu.CompilerParams(
            needs_layout_passes=False,
            disable_bounds_checks=True,
            disable_semaphore_checks=True,
        ),
        out_type=jax.ShapeDtypeStruct((_BC * _L,), jnp.float32),
        scratch_types=[
            pltpu.VMEM((_RPW * _N,), jnp.float32),
            pltpu.VMEM((_C * _L,), jnp.float32),
            pltpu.VMEM((_RPW * _L,), jnp.float32),
            pltpu.SemaphoreType.DMA,
            pltpu.SemaphoreType.DMA,
            pltpu.SemaphoreType.DMA,
            pltpu.SemaphoreType.DMA,
        ],
    )(_sc_topk_body)


def kernel(x, W1, b1, W2, b2):
    w2p = jnp.concatenate(
        [W2, (b2 + b1[0] * jnp.sum(W2, axis=1))[:, None],
         jnp.zeros((_C, _L - 2 * _R - 1), jnp.float32)], axis=1).reshape(-1)
    sc = _sc_topk()
    outs = []
    for i in range(_NSPLIT):
        s = _scores(x, W1, i * _BC)
        outs.append(sc(s.reshape(-1), w2p))
    out16 = jnp.concatenate(outs)
    return out16.reshape(_B, _L)[:, :_C]


# speedup vs baseline: 1.0082x; 1.0082x over previous
import functools

import jax
import jax.numpy as jnp
from jax import lax
from jax.experimental import pallas as pl
from jax.experimental.pallas import tpu as pltpu
from jax.experimental.pallas import tpu_sc as plsc

_B, _N, _D, _R, _C = 64, 8192, 128, 5, 5
_L = 16
_NW = 32
_NSPLIT = 1
_BC = _B // _NSPLIT
_RPW = _BC // _NW


_RB = 2


def _score_body(xa_ref, xb_ref, w_ref, o_ref):
    w = w_ref[...]
    for r in range(_RB):
        o_ref[r, 0, : _N // 2] = lax.dot_general(
            w, xa_ref[r], (((1,), (1,)), ((), ())),
            preferred_element_type=jnp.float32)[0]
        o_ref[r, 0, _N // 2 :] = lax.dot_general(
            w, xb_ref[r], (((1,), (1,)), ((), ())),
            preferred_element_type=jnp.float32)[0]


def _scores(x, w1row, off):
    return pl.pallas_call(
        _score_body,
        grid=(_BC // _RB,),
        in_specs=[
            pl.BlockSpec((_RB, _N // 2, _D), lambda b: (b + off, 0, 0)),
            pl.BlockSpec((_RB, _N // 2, _D), lambda b: (b + off, 1, 0)),
            pl.BlockSpec((1, _D), lambda b: (0, 0)),
        ],
        out_specs=pl.BlockSpec((_RB, 1, _N), lambda b: (b, 0, 0)),
        out_shape=jax.ShapeDtypeStruct((_BC, 1, _N), jnp.float32),
    )(x, x, w1row)


def _insert(lists, v):
    t0, t1, t2, t3, t4, u0, u1, u2, u3, u4 = lists
    a = v
    m = jnp.maximum(t0, a); a = jnp.minimum(t0, a); t0 = m
    m = jnp.maximum(t1, a); a = jnp.minimum(t1, a); t1 = m
    m = jnp.maximum(t2, a); a = jnp.minimum(t2, a); t2 = m
    m = jnp.maximum(t3, a); a = jnp.minimum(t3, a); t3 = m
    t4 = jnp.maximum(t4, a)
    b = v
    m = jnp.minimum(u0, b); b = jnp.maximum(u0, b); u0 = m
    m = jnp.minimum(u1, b); b = jnp.maximum(u1, b); u1 = m
    m = jnp.minimum(u2, b); b = jnp.maximum(u2, b); u2 = m
    m = jnp.minimum(u3, b); b = jnp.maximum(u3, b); u3 = m
    u4 = jnp.minimum(u4, b)
    return (t0, t1, t2, t3, t4, u0, u1, u2, u3, u4)


def _extract_feat(lists, lane, neg, pos, zeros):
    t0, t1, t2, t3, t4, u0, u1, u2, u3, u4 = lists
    fv = jnp.where(lane == 2 * _R, jnp.float32(1.0), zeros)
    for k in range(_R):
        m = jnp.max(t0)
        fv = jnp.where(lane == k, m, fv)
        sel = lane == plsc.all_reduce_ffs(t0 == m)
        t0 = jnp.where(sel, t1, t0)
        t1 = jnp.where(sel, t2, t1)
        t2 = jnp.where(sel, t3, t2)
        t3 = jnp.where(sel, t4, t3)
        t4 = jnp.where(sel, neg, t4)
    for k in range(_R):
        m = jnp.min(u0)
        fv = jnp.where(lane == 2 * _R - 1 - k, m, fv)
        sel = lane == plsc.all_reduce_ffs(u0 == m)
        u0 = jnp.where(sel, u1, u0)
        u1 = jnp.where(sel, u2, u1)
        u2 = jnp.where(sel, u3, u2)
        u3 = jnp.where(sel, u4, u3)
        u4 = jnp.where(sel, pos, u4)
    return fv


_NHALF = _N // 2


def _sc_topk_body(s_hbm, w2_hbm, out_hbm, s_v, w2_v, out_v,
                  sem0, sem1, sem2, sem3):
    cid = lax.axis_index("c")
    sid = lax.axis_index("s")
    wid = sid * 2 + cid
    base = wid * (_RPW * _N)
    c0 = pltpu.async_copy(s_hbm.at[pl.ds(base, _NHALF)],
                          s_v.at[pl.ds(0, _NHALF)], sem0)
    c1 = pltpu.async_copy(s_hbm.at[pl.ds(base + _N, _NHALF)],
                          s_v.at[pl.ds(_N, _NHALF)], sem1)
    c2 = pltpu.async_copy(s_hbm.at[pl.ds(base + _NHALF, _NHALF)],
                          s_v.at[pl.ds(_NHALF, _NHALF)], sem2)
    c3 = pltpu.async_copy(s_hbm.at[pl.ds(base + _N + _NHALF, _NHALF)],
                          s_v.at[pl.ds(_N + _NHALF, _NHALF)], sem3)
    pltpu.sync_copy(w2_hbm, w2_v)

    neg = jnp.full((_L,), -jnp.inf, jnp.float32)
    pos = jnp.full((_L,), jnp.inf, jnp.float32)
    zeros = jnp.zeros((_L,), jnp.float32)
    lane = lax.iota(jnp.int32, _L)

    def step(i, carry):
        r0, r1 = carry
        v0 = s_v[pl.ds(i * _L, _L)]
        v1 = s_v[pl.ds(_N + i * _L, _L)]
        return (_insert(r0, v0), _insert(r1, v1))

    init = ((neg,) * 5 + (pos,) * 5, (neg,) * 5 + (pos,) * 5)
    c0.wait()
    c1.wait()
    carry = lax.fori_loop(0, _NHALF // _L, step, init, unroll=4)
    c2.wait()
    c3.wait()
    carry = lax.fori_loop(_NHALF // _L, _N // _L, step, carry, unroll=4)

    for rl in range(_RPW):
        fv = _extract_feat(carry[rl], lane, neg, pos, zeros)
        ov = zeros
        for c in range(_C):
            w = w2_v[pl.ds(c * _L, _L)]
            ov = jnp.where(lane == c, jnp.sum(fv * w), ov)
        out_v[pl.ds(rl * _L, _L)] = ov

    pltpu.sync_copy(out_v, out_hbm.at[pl.ds(wid * (_RPW * _L), _RPW * _L)])


@functools.cache
def _sc_topk():
    return functools.partial(
        pl.kernel,
        mesh=plsc.VectorSubcoreMesh(core_axis_name="c", subcore_axis_name="s"),
        compiler_params=pltpu.CompilerParams(
            needs_layout_passes=False,
            disable_bounds_checks=True,
            disable_semaphore_checks=True,
        ),
        out_type=jax.ShapeDtypeStruct((_BC * _L,), jnp.float32),
        scratch_types=[
            pltpu.VMEM((_RPW * _N,), jnp.float32),
            pltpu.VMEM((_C * _L,), jnp.float32),
            pltpu.VMEM((_RPW * _L,), jnp.float32),
            pltpu.SemaphoreType.DMA,
            pltpu.SemaphoreType.DMA,
            pltpu.SemaphoreType.DMA,
            pltpu.SemaphoreType.DMA,
        ],
    )(_sc_topk_body)


def kernel(x, W1, b1, W2, b2):
    w2p = jnp.concatenate(
        [W2, (b2 + b1[0] * jnp.sum(W2, axis=1))[:, None],
         jnp.zeros((_C, _L - 2 * _R - 1), jnp.float32)], axis=1).reshape(-1)
    sc = _sc_topk()
    outs = []
    for i in range(_NSPLIT):
        s = _scores(x, W1, i * _BC)
        outs.append(sc(s.reshape(-1), w2p))
    out16 = jnp.concatenate(outs)
    return out16.reshape(_B, _L)[:, :_C]
